# baseline (speedup 1.0000x reference)
import functools

import jax
import jax.numpy as jnp
from jax import lax
from jax.experimental import pallas as pl
from jax.experimental.pallas import tpu as pltpu
from jax.experimental.pallas import tpu_sc as plsc

N_NODES = 10000
D_FEAT = 256
UNITS = 256
N_EDGES = 160000

NC = 2
NS = 16
NW = NC * NS
NODES_PER = 320
NPAD = NW * NODES_PER
CHUNK = 64
NGRP = CHUNK // 16
EPAD = N_EDGES + 256
SRC_PAD = NPAD + 8
AWIN = 384
APAD = NPAD + 64
HWORDS = UNITS // 2
NBLK = HWORDS // 16
JBLK = UNITS // 16


def _tc_body(x_ref, w_ref, ka_ref, h_ref, ab_ref):
    h = jnp.dot(x_ref[...], w_ref[...], preferred_element_type=jnp.float32)
    lo = lax.bitcast_convert_type(
        h[:, :128].astype(jnp.bfloat16), jnp.int16).astype(jnp.int32)
    hi = lax.bitcast_convert_type(
        h[:, 128:].astype(jnp.bfloat16), jnp.int16).astype(jnp.int32)
    h_ref[...] = jnp.bitwise_or(
        jnp.bitwise_and(lo, jnp.int32(0xFFFF)),
        jnp.left_shift(hi, 16))
    ab_ref[...] = lax.dot_general(
        h, ka_ref[...], (((1,), (1,)), ((), ())),
        preferred_element_type=jnp.float32)


def _tc_transform(x, w, ka_pad):
    rows = 2000
    return pl.pallas_call(
        _tc_body,
        grid=(N_NODES // rows,),
        in_specs=[
            pl.BlockSpec((rows, D_FEAT), lambda i: (i, 0)),
            pl.BlockSpec((D_FEAT, UNITS), lambda i: (0, 0)),
            pl.BlockSpec((128, D_FEAT), lambda i: (0, 0)),
        ],
        out_specs=[
            pl.BlockSpec((rows, HWORDS), lambda i: (i, 0)),
            pl.BlockSpec((rows, 128), lambda i: (i, 0)),
        ],
        out_shape=[
            jax.ShapeDtypeStruct((N_NODES, HWORDS), jnp.int32),
            jax.ShapeDtypeStruct((N_NODES, 128), jnp.float32),
        ],
    )(x, w, ka_pad)


def _sc_aggregate(h32, a_p, b_p, src_p, dst_p, bounds_p, bias):
    mesh = plsc.VectorSubcoreMesh(core_axis_name="c", subcore_axis_name="s")

    @functools.partial(
        pl.kernel,
        mesh=mesh,
        out_type=jax.ShapeDtypeStruct((N_NODES, UNITS), jnp.float32),
        compiler_params=pltpu.CompilerParams(needs_layout_passes=False),
        scratch_types=[
            pltpu.VMEM((NODES_PER, UNITS), jnp.float32),
            pltpu.VMEM((AWIN,), jnp.float32),
            pltpu.VMEM((NPAD,), jnp.float32),
            pltpu.VMEM((UNITS,), jnp.float32),
            pltpu.VMEM((40,), jnp.int32),
            pltpu.VMEM((CHUNK + 16,), jnp.int32),
            pltpu.VMEM((CHUNK + 16,), jnp.int32),
            pltpu.VMEM((CHUNK,), jnp.int32),
            pltpu.VMEM((CHUNK,), jnp.int32),
            pltpu.VMEM((CHUNK,), jnp.float32),
            pltpu.VMEM((CHUNK,), jnp.float32),
            pltpu.VMEM((CHUNK, HWORDS), jnp.int32),
            pltpu.VMEM((CHUNK, HWORDS), jnp.int32),
            pltpu.VMEM((UNITS,), jnp.float32),
            pltpu.VMEM((16,), jnp.float32),
            pltpu.VMEM((16,), jnp.int32),
            pltpu.SemaphoreType.DMA,
            pltpu.SemaphoreType.DMA,
            pltpu.SemaphoreType.DMA,
            pltpu.SemaphoreType.DMA,
        ],
    )
    def body(h_hbm, a_hbm, b_hbm, src_hbm, dst_hbm, bounds_hbm, bias_hbm,
             out_hbm, stage, a_v, b_v, bias_v, bounds_v,
             src_a, src_b, dst_a, dst_b, sco_a, sco_b, rows_a, rows_b,
             snap_v, dsnap_v, csrc_v, sem_ia, sem_ib, sem_ra, sem_rb):
        wid = lax.axis_index("s") * NC + lax.axis_index("c")
        n_lo = wid * NODES_PER
        a_base = pl.multiple_of(jnp.maximum(n_lo - 32, 0), 8)

        pltpu.sync_copy(a_hbm.at[pl.ds(a_base, AWIN)], a_v)
        pltpu.sync_copy(b_hbm, b_v)
        pltpu.sync_copy(bias_hbm, bias_v)
        pltpu.sync_copy(bounds_hbm, bounds_v)

        iota = lax.broadcasted_iota(jnp.int32, (16,), 0)
        bitval = jnp.left_shift(jnp.ones((16,), jnp.int32), iota)
        n_lo_v = jnp.full((16,), n_lo, jnp.int32)
        n_hi_v = n_lo_v + NODES_PER
        a_base_v = jnp.full((16,), a_base, jnp.int32)
        cols = []
        for j8 in range(NBLK):
            cols.append(iota + 16 * j8)
            cols.append(iota + 128 + 16 * j8)

        def prefill(i, _):
            for j in range(JBLK):
                stage[i, pl.ds(j * 16, 16)] = bias_v[pl.ds(j * 16, 16)]
            return 0
        lax.fori_loop(0, NODES_PER, prefill, 0)

        widv = jnp.full((16,), wid, jnp.int32)
        e_lo = jnp.max(plsc.load_gather(bounds_v, [widv]))
        e_hi = jnp.max(plsc.load_gather(bounds_v, [widv + 1]))
        e0 = (e_lo // 8) * 8
        nchunks = (e_hi - e0 + (CHUNK - 1)) // CHUNK
        pairs = jnp.maximum((nchunks + 1) // 2, 1)

        def ids_start(c, src_ref, dst_ref, sem):
            e = pl.multiple_of(e0 + c * CHUNK, 8)
            pltpu.async_copy(src_hbm.at[pl.ds(e, CHUNK)],
                             src_ref.at[pl.ds(16, CHUNK)], sem)
            pltpu.async_copy(dst_hbm.at[pl.ds(e, CHUNK)], dst_ref, sem)

        def ids_wait(src_ref, dst_ref, sem):
            pltpu.make_async_copy(src_hbm.at[pl.ds(0, CHUNK)],
                                  src_ref.at[pl.ds(16, CHUNK)], sem).wait()
            pltpu.make_async_copy(dst_hbm.at[pl.ds(0, CHUNK)],
                                  dst_ref, sem).wait()

        def rows_start(dst_ref, rows_ref, sem):
            pltpu.async_copy(h_hbm.at[dst_ref], rows_ref, sem)

        def rows_wait(dst_ref, rows_ref, sem):
            pltpu.make_async_copy(h_hbm.at[dst_ref], rows_ref, sem).wait()

        def flush_stores(acc, accden, new_src):
            old_src = csrc_v[pl.ds(0, 16)]
            nloc = old_src - n_lo_v
            vmask = (old_src >= n_lo_v) & (old_src < n_hi_v)
            w = 1.0 / (accden - dsnap_v[pl.ds(0, 16)])
            for j in range(JBLK):
                seg = acc[j] - snap_v[pl.ds(j * 16, 16)]
                plsc.store_scatter(
                    stage, [nloc, cols[j]],
                    seg * w + plsc.load_gather(bias_v, [cols[j]]),
                    mask=vmask)
                snap_v[pl.ds(j * 16, 16)] = acc[j]
            dsnap_v[pl.ds(0, 16)] = accden
            csrc_v[pl.ds(0, 16)] = new_src

        def process(src_ref, dst_ref, sco_ref, rows_ref, carry):
            for g in range(NGRP):
                s16 = src_ref[pl.ds(16 + g * 16, 16)]
                d16 = dst_ref[pl.ds(g * 16, 16)]
                prevv = plsc.load_gather(src_ref, [iota + (15 + g * 16)])
                aidx = jnp.clip(s16 - a_base_v, 0, AWIN - 1)
                x = plsc.load_gather(a_v, [aidx]) + plsc.load_gather(b_v, [d16])
                x = jnp.where(x >= 0.0, x, 0.2 * x)
                sco_ref[pl.ds(g * 16, 16)] = jnp.exp(x)
                bmask = jnp.sum(jnp.where(s16 != prevv, bitval, 0))

                def edge_body(i, car):
                    acc, accden = car
                    idx = g * 16 + i
                    s_vec = plsc.load_gather(
                        sco_ref, [jnp.full((16,), idx, jnp.int32)])
                    bit = jnp.bitwise_and(
                        jnp.right_shift(bmask, i), jnp.int32(1))

                    @pl.when(bit == 1)
                    def _():
                        new_src = plsc.load_gather(
                            src_ref, [jnp.full((16,), 16 + idx, jnp.int32)])
                        flush_stores(acc, accden, new_src)

                    acc2 = []
                    for j8 in range(NBLK):
                        v = rows_ref[idx, pl.ds(j8 * 16, 16)]
                        lo = plsc.bitcast(jnp.left_shift(v, 16), jnp.float32)
                        hi = plsc.bitcast(
                            jnp.bitwise_and(v, jnp.int32(-65536)), jnp.float32)
                        acc2.append(acc[2 * j8] + s_vec * lo)
                        acc2.append(acc[2 * j8 + 1] + s_vec * hi)
                    return tuple(acc2), accden + s_vec

                carry = lax.fori_loop(0, 16, edge_body, carry)
            return carry

        acc0 = tuple(jnp.zeros((16,), jnp.float32) for _ in range(JBLK))
        den0 = jnp.zeros((16,), jnp.float32)
        for j in range(JBLK):
            snap_v[pl.ds(j * 16, 16)] = jnp.zeros((16,), jnp.float32)
        dsnap_v[pl.ds(0, 16)] = jnp.zeros((16,), jnp.float32)
        csrc_v[pl.ds(0, 16)] = jnp.full((16,), -1, jnp.int32)

        src_a[pl.ds(0, 16)] = jnp.full((16,), -1, jnp.int32)
        src_b[pl.ds(0, 16)] = jnp.full((16,), -1, jnp.int32)
        ids_start(0, src_a, dst_a, sem_ia)
        ids_wait(src_a, dst_a, sem_ia)
        rows_start(dst_a, rows_a, sem_ra)
        ids_start(1, src_b, dst_b, sem_ib)

        def pair_body(m, carry):
            ids_wait(src_b, dst_b, sem_ib)
            src_b[pl.ds(0, 16)] = src_a[pl.ds(CHUNK, 16)]
            rows_start(dst_b, rows_b, sem_rb)
            rows_wait(dst_a, rows_a, sem_ra)
            carry = process(src_a, dst_a, sco_a, rows_a, carry)

            @pl.when(m + 1 < pairs)
            def _():
                ids_start(2 * m + 2, src_a, dst_a, sem_ia)

            rows_wait(dst_b, rows_b, sem_rb)
            carry = process(src_b, dst_b, sco_b, rows_b, carry)

            @pl.when(m + 1 < pairs)
            def _():
                ids_wait(src_a, dst_a, sem_ia)
                src_a[pl.ds(0, 16)] = src_b[pl.ds(CHUNK, 16)]
                rows_start(dst_a, rows_a, sem_ra)
                ids_start(2 * m + 3, src_b, dst_b, sem_ib)

            return carry

        acc, accden = lax.fori_loop(0, pairs, pair_body, (acc0, den0))

        flush_stores(acc, accden, jnp.full((16,), -1, jnp.int32))

        last_rows = N_NODES - (NW - 1) * NODES_PER

        @pl.when(wid < NW - 1)
        def _():
            pltpu.sync_copy(stage, out_hbm.at[pl.ds(n_lo, NODES_PER)])

        @pl.when(wid == NW - 1)
        def _():
            pltpu.sync_copy(
                stage.at[pl.ds(0, last_rows)],
                out_hbm.at[pl.ds((NW - 1) * NODES_PER, last_rows)])

    return body(h32, a_p, b_p, src_p, dst_p, bounds_p, bias)


def kernel(node_states, edges, kernel, bias, kernel_attention, training):
    del training
    x = node_states.astype(jnp.float32)
    w = kernel.astype(jnp.float32)
    ka = kernel_attention.astype(jnp.float32)
    ka_pad = jnp.zeros((128, D_FEAT), jnp.float32)
    ka_pad = ka_pad.at[0].set(ka[:UNITS]).at[1].set(ka[UNITS:])

    h32, ab = _tc_transform(x, w, ka_pad)

    src = edges[:, 0].astype(jnp.int32)
    dst = edges[:, 1].astype(jnp.int32)
    src_p = jnp.concatenate(
        [src, jnp.full((EPAD - N_EDGES,), SRC_PAD, jnp.int32)])
    dst_p = jnp.concatenate(
        [dst, jnp.zeros((EPAD - N_EDGES,), jnp.int32)])
    bound_nodes = jnp.arange(33, dtype=jnp.int32) * NODES_PER
    bounds = jnp.searchsorted(src, bound_nodes).astype(jnp.int32)
    bounds_p = jnp.concatenate([bounds, jnp.zeros((7,), jnp.int32)])
    a_p = jnp.concatenate([ab[:, 0], jnp.zeros((APAD - N_NODES,), jnp.float32)])
    b_p = jnp.concatenate([ab[:, 1], jnp.zeros((NPAD - N_NODES,), jnp.float32)])

    return _sc_aggregate(h32, a_p, b_p, src_p, dst_p, bounds_p,
                         bias.astype(jnp.float32))


# speedup vs baseline: 1.0973x; 1.0973x over previous
import functools

import jax
import jax.numpy as jnp
from jax import lax
from jax.experimental import pallas as pl
from jax.experimental.pallas import tpu as pltpu
from jax.experimental.pallas import tpu_sc as plsc

N_NODES = 10000
D_FEAT = 256
UNITS = 256
N_EDGES = 160000

NC = 2
NS = 16
NW = NC * NS
NODES_PER = 320
NPAD = NW * NODES_PER
CHUNK = 64
NGRP = CHUNK // 16
EPAD = N_EDGES + 256
SRC_PAD = NPAD + 8
AWIN = 384
APAD = NPAD + 64
HWORDS = UNITS // 2
NBLK = HWORDS // 16
JBLK = UNITS // 16


def _tc_body(x_ref, w_ref, ka_ref, h_ref, ab_ref):
    h = jnp.dot(x_ref[...], w_ref[...], preferred_element_type=jnp.float32)
    lo = lax.bitcast_convert_type(
        h[:, :128].astype(jnp.bfloat16), jnp.int16).astype(jnp.int32)
    hi = lax.bitcast_convert_type(
        h[:, 128:].astype(jnp.bfloat16), jnp.int16).astype(jnp.int32)
    h_ref[...] = jnp.bitwise_or(
        jnp.bitwise_and(lo, jnp.int32(0xFFFF)),
        jnp.left_shift(hi, 16))
    ab_ref[...] = lax.dot_general(
        h, ka_ref[...], (((1,), (1,)), ((), ())),
        preferred_element_type=jnp.float32)


def _tc_transform(x, w, ka_pad):
    rows = 2000
    return pl.pallas_call(
        _tc_body,
        grid=(N_NODES // rows,),
        in_specs=[
            pl.BlockSpec((rows, D_FEAT), lambda i: (i, 0)),
            pl.BlockSpec((D_FEAT, UNITS), lambda i: (0, 0)),
            pl.BlockSpec((128, D_FEAT), lambda i: (0, 0)),
        ],
        out_specs=[
            pl.BlockSpec((rows, HWORDS), lambda i: (i, 0)),
            pl.BlockSpec((rows, 128), lambda i: (i, 0)),
        ],
        out_shape=[
            jax.ShapeDtypeStruct((N_NODES, HWORDS), jnp.int32),
            jax.ShapeDtypeStruct((N_NODES, 128), jnp.float32),
        ],
    )(x, w, ka_pad)


def _sc_aggregate(h32, a_p, b_p, src_p, dst_p, bounds_p, bias):
    mesh = plsc.VectorSubcoreMesh(core_axis_name="c", subcore_axis_name="s")

    @functools.partial(
        pl.kernel,
        mesh=mesh,
        out_type=jax.ShapeDtypeStruct((N_NODES, UNITS), jnp.float32),
        compiler_params=pltpu.CompilerParams(needs_layout_passes=False),
        scratch_types=[
            pltpu.VMEM((NODES_PER, UNITS), jnp.float32),
            pltpu.VMEM((AWIN,), jnp.float32),
            pltpu.VMEM((NPAD,), jnp.float32),
            pltpu.VMEM((UNITS,), jnp.float32),
            pltpu.VMEM((40,), jnp.int32),
            pltpu.VMEM((CHUNK + 16,), jnp.int32),
            pltpu.VMEM((CHUNK + 16,), jnp.int32),
            pltpu.VMEM((CHUNK,), jnp.int32),
            pltpu.VMEM((CHUNK,), jnp.int32),
            pltpu.VMEM((CHUNK,), jnp.float32),
            pltpu.VMEM((CHUNK,), jnp.float32),
            pltpu.VMEM((CHUNK, HWORDS), jnp.int32),
            pltpu.VMEM((CHUNK, HWORDS), jnp.int32),
            pltpu.VMEM((UNITS,), jnp.float32),
            pltpu.VMEM((16,), jnp.float32),
            pltpu.VMEM((16,), jnp.int32),
            pltpu.SemaphoreType.DMA,
            pltpu.SemaphoreType.DMA,
            pltpu.SemaphoreType.DMA,
            pltpu.SemaphoreType.DMA,
        ],
    )
    def body(h_hbm, a_hbm, b_hbm, src_hbm, dst_hbm, bounds_hbm, bias_hbm,
             out_hbm, stage, a_v, b_v, bias_v, bounds_v,
             src_a, src_b, dst_a, dst_b, sco_a, sco_b, rows_a, rows_b,
             snap_v, dsnap_v, csrc_v, sem_ia, sem_ib, sem_ra, sem_rb):
        wid = lax.axis_index("s") * NC + lax.axis_index("c")
        n_lo = wid * NODES_PER
        a_base = pl.multiple_of(jnp.maximum(n_lo - 32, 0), 8)

        pltpu.sync_copy(a_hbm.at[pl.ds(a_base, AWIN)], a_v)
        pltpu.sync_copy(b_hbm, b_v)
        pltpu.sync_copy(bias_hbm, bias_v)
        pltpu.sync_copy(bounds_hbm, bounds_v)

        iota = lax.broadcasted_iota(jnp.int32, (16,), 0)
        bitval = jnp.left_shift(jnp.ones((16,), jnp.int32), iota)
        n_lo_v = jnp.full((16,), n_lo, jnp.int32)
        n_hi_v = n_lo_v + NODES_PER
        a_base_v = jnp.full((16,), a_base, jnp.int32)
        cols = []
        for j8 in range(NBLK):
            cols.append(iota + 16 * j8)
            cols.append(iota + 128 + 16 * j8)

        def prefill(i, _):
            for j in range(JBLK):
                stage[i, pl.ds(j * 16, 16)] = bias_v[pl.ds(j * 16, 16)]
            return 0
        lax.fori_loop(0, NODES_PER, prefill, 0)

        widv = jnp.full((16,), wid, jnp.int32)
        e_lo = jnp.max(plsc.load_gather(bounds_v, [widv]))
        e_hi = jnp.max(plsc.load_gather(bounds_v, [widv + 1]))
        e0 = (e_lo // 8) * 8
        nchunks = (e_hi - e0 + (CHUNK - 1)) // CHUNK
        pairs = jnp.maximum((nchunks + 1) // 2, 1)

        def ids_start(c, src_ref, dst_ref, sem):
            e = pl.multiple_of(e0 + c * CHUNK, 8)
            pltpu.async_copy(src_hbm.at[pl.ds(e, CHUNK)],
                             src_ref.at[pl.ds(16, CHUNK)], sem)
            pltpu.async_copy(dst_hbm.at[pl.ds(e, CHUNK)], dst_ref, sem)

        def ids_wait(src_ref, dst_ref, sem):
            pltpu.make_async_copy(src_hbm.at[pl.ds(0, CHUNK)],
                                  src_ref.at[pl.ds(16, CHUNK)], sem).wait()
            pltpu.make_async_copy(dst_hbm.at[pl.ds(0, CHUNK)],
                                  dst_ref, sem).wait()

        def rows_start(dst_ref, rows_ref, sem):
            pltpu.async_copy(h_hbm.at[dst_ref], rows_ref, sem)

        def rows_wait(dst_ref, rows_ref, sem):
            pltpu.make_async_copy(h_hbm.at[dst_ref], rows_ref, sem).wait()

        def flush_stores(acc, accden, new_src):
            old_src = csrc_v[pl.ds(0, 16)]
            nloc = old_src - n_lo_v
            vmask = (old_src >= n_lo_v) & (old_src < n_hi_v)
            w = 1.0 / (accden - dsnap_v[pl.ds(0, 16)])
            for j in range(JBLK):
                seg = acc[j] - snap_v[pl.ds(j * 16, 16)]
                plsc.store_scatter(
                    stage, [nloc, cols[j]],
                    seg * w + plsc.load_gather(bias_v, [cols[j]]),
                    mask=vmask)
                snap_v[pl.ds(j * 16, 16)] = acc[j]
            dsnap_v[pl.ds(0, 16)] = accden
            csrc_v[pl.ds(0, 16)] = new_src

        def process(src_ref, dst_ref, sco_ref, rows_ref, carry):
            for g in range(NGRP):
                s16 = src_ref[pl.ds(16 + g * 16, 16)]
                d16 = dst_ref[pl.ds(g * 16, 16)]
                prevv = plsc.load_gather(src_ref, [iota + (15 + g * 16)])
                aidx = jnp.clip(s16 - a_base_v, 0, AWIN - 1)
                x = plsc.load_gather(a_v, [aidx]) + plsc.load_gather(b_v, [d16])
                x = jnp.where(x >= 0.0, x, 0.2 * x)
                sco_ref[pl.ds(g * 16, 16)] = jnp.exp(x)
                bmask = jnp.sum(jnp.where(s16 != prevv, bitval, 0))

                def edge_body(i, car):
                    acc, accden = car
                    idx = g * 16 + i
                    s_vec = plsc.load_gather(
                        sco_ref, [jnp.full((16,), idx, jnp.int32)])
                    bit = jnp.bitwise_and(
                        jnp.right_shift(bmask, i), jnp.int32(1))

                    acc2 = []
                    for j8 in range(NBLK):
                        v = rows_ref[idx, pl.ds(j8 * 16, 16)]
                        lo = plsc.bitcast(jnp.left_shift(v, 16), jnp.float32)
                        hi = plsc.bitcast(
                            jnp.bitwise_and(v, jnp.int32(-65536)), jnp.float32)
                        acc2.append(acc[2 * j8] + s_vec * lo)
                        acc2.append(acc[2 * j8 + 1] + s_vec * hi)

                    @pl.when(bit == 1)
                    def _():
                        new_src = plsc.load_gather(
                            src_ref, [jnp.full((16,), 16 + idx, jnp.int32)])
                        flush_stores(acc, accden, new_src)

                    return tuple(acc2), accden + s_vec

                carry = lax.fori_loop(0, 16, edge_body, carry)
            return carry

        acc0 = tuple(jnp.zeros((16,), jnp.float32) for _ in range(JBLK))
        den0 = jnp.zeros((16,), jnp.float32)
        for j in range(JBLK):
            snap_v[pl.ds(j * 16, 16)] = jnp.zeros((16,), jnp.float32)
        dsnap_v[pl.ds(0, 16)] = jnp.zeros((16,), jnp.float32)
        csrc_v[pl.ds(0, 16)] = jnp.full((16,), -1, jnp.int32)

        src_a[pl.ds(0, 16)] = jnp.full((16,), -1, jnp.int32)
        src_b[pl.ds(0, 16)] = jnp.full((16,), -1, jnp.int32)
        ids_start(0, src_a, dst_a, sem_ia)
        ids_wait(src_a, dst_a, sem_ia)
        rows_start(dst_a, rows_a, sem_ra)
        ids_start(1, src_b, dst_b, sem_ib)

        def pair_body(m, carry):
            ids_wait(src_b, dst_b, sem_ib)
            src_b[pl.ds(0, 16)] = src_a[pl.ds(CHUNK, 16)]
            rows_start(dst_b, rows_b, sem_rb)
            rows_wait(dst_a, rows_a, sem_ra)
            carry = process(src_a, dst_a, sco_a, rows_a, carry)

            @pl.when(m + 1 < pairs)
            def _():
                ids_start(2 * m + 2, src_a, dst_a, sem_ia)

            rows_wait(dst_b, rows_b, sem_rb)
            carry = process(src_b, dst_b, sco_b, rows_b, carry)

            @pl.when(m + 1 < pairs)
            def _():
                ids_wait(src_a, dst_a, sem_ia)
                src_a[pl.ds(0, 16)] = src_b[pl.ds(CHUNK, 16)]
                rows_start(dst_a, rows_a, sem_ra)
                ids_start(2 * m + 3, src_b, dst_b, sem_ib)

            return carry

        acc, accden = lax.fori_loop(0, pairs, pair_body, (acc0, den0))

        flush_stores(acc, accden, jnp.full((16,), -1, jnp.int32))

        last_rows = N_NODES - (NW - 1) * NODES_PER

        @pl.when(wid < NW - 1)
        def _():
            pltpu.sync_copy(stage, out_hbm.at[pl.ds(n_lo, NODES_PER)])

        @pl.when(wid == NW - 1)
        def _():
            pltpu.sync_copy(
                stage.at[pl.ds(0, last_rows)],
                out_hbm.at[pl.ds((NW - 1) * NODES_PER, last_rows)])

    return body(h32, a_p, b_p, src_p, dst_p, bounds_p, bias)


def kernel(node_states, edges, kernel, bias, kernel_attention, training):
    del training
    x = node_states.astype(jnp.float32)
    w = kernel.astype(jnp.float32)
    ka = kernel_attention.astype(jnp.float32)
    ka_pad = jnp.zeros((128, D_FEAT), jnp.float32)
    ka_pad = ka_pad.at[0].set(ka[:UNITS]).at[1].set(ka[UNITS:])

    h32, ab = _tc_transform(x, w, ka_pad)

    src = edges[:, 0].astype(jnp.int32)
    dst = edges[:, 1].astype(jnp.int32)
    src_p = jnp.concatenate(
        [src, jnp.full((EPAD - N_EDGES,), SRC_PAD, jnp.int32)])
    dst_p = jnp.concatenate(
        [dst, jnp.zeros((EPAD - N_EDGES,), jnp.int32)])
    bound_nodes = jnp.arange(33, dtype=jnp.int32) * NODES_PER
    bounds = jnp.searchsorted(src, bound_nodes).astype(jnp.int32)
    bounds_p = jnp.concatenate([bounds, jnp.zeros((7,), jnp.int32)])
    a_p = jnp.concatenate([ab[:, 0], jnp.zeros((APAD - N_NODES,), jnp.float32)])
    b_p = jnp.concatenate([ab[:, 1], jnp.zeros((NPAD - N_NODES,), jnp.float32)])

    return _sc_aggregate(h32, a_p, b_p, src_p, dst_p, bounds_p,
                         bias.astype(jnp.float32))


# speedup vs baseline: 1.1557x; 1.0532x over previous
import functools

import jax
import jax.numpy as jnp
from jax import lax
from jax.experimental import pallas as pl
from jax.experimental.pallas import tpu as pltpu
from jax.experimental.pallas import tpu_sc as plsc

N_NODES = 10000
D_FEAT = 256
UNITS = 256
N_EDGES = 160000

NC = 2
NS = 16
NW = NC * NS
NODES_PER = 320
NPAD = NW * NODES_PER
CHUNK = 64
NGRP = CHUNK // 16
EPAD = N_EDGES + 256
SRC_PAD = NPAD + 8
AWIN = 384
APAD = NPAD + 64
HWORDS = UNITS // 2
NBLK = HWORDS // 16
JBLK = UNITS // 16


def _tc_body(x_ref, w_ref, ka_ref, h_ref, ab_ref):
    h = jnp.dot(x_ref[...], w_ref[...], preferred_element_type=jnp.float32)
    lo = lax.bitcast_convert_type(
        h[:, :128].astype(jnp.bfloat16), jnp.int16).astype(jnp.int32)
    hi = lax.bitcast_convert_type(
        h[:, 128:].astype(jnp.bfloat16), jnp.int16).astype(jnp.int32)
    h_ref[...] = jnp.bitwise_or(
        jnp.bitwise_and(lo, jnp.int32(0xFFFF)),
        jnp.left_shift(hi, 16))
    ab_ref[...] = lax.dot_general(
        h, ka_ref[...], (((1,), (1,)), ((), ())),
        preferred_element_type=jnp.float32)


def _tc_transform(x, w, ka_pad):
    rows = 2000
    return pl.pallas_call(
        _tc_body,
        grid=(N_NODES // rows,),
        in_specs=[
            pl.BlockSpec((rows, D_FEAT), lambda i: (i, 0)),
            pl.BlockSpec((D_FEAT, UNITS), lambda i: (0, 0)),
            pl.BlockSpec((128, D_FEAT), lambda i: (0, 0)),
        ],
        out_specs=[
            pl.BlockSpec((rows, HWORDS), lambda i: (i, 0)),
            pl.BlockSpec((rows, 128), lambda i: (i, 0)),
        ],
        out_shape=[
            jax.ShapeDtypeStruct((N_NODES, HWORDS), jnp.int32),
            jax.ShapeDtypeStruct((N_NODES, 128), jnp.float32),
        ],
    )(x, w, ka_pad)


def _sc_aggregate(h32, a_p, b_p, src_p, dst_p, bounds_p, bias):
    mesh = plsc.VectorSubcoreMesh(core_axis_name="c", subcore_axis_name="s")

    @functools.partial(
        pl.kernel,
        mesh=mesh,
        out_type=jax.ShapeDtypeStruct((N_NODES, UNITS), jnp.float32),
        compiler_params=pltpu.CompilerParams(needs_layout_passes=False),
        scratch_types=[
            pltpu.VMEM((NODES_PER, UNITS), jnp.float32),
            pltpu.VMEM((AWIN,), jnp.float32),
            pltpu.VMEM((NPAD,), jnp.float32),
            pltpu.VMEM((UNITS,), jnp.float32),
            pltpu.VMEM((40,), jnp.int32),
            pltpu.VMEM((CHUNK + 16,), jnp.int32),
            pltpu.VMEM((CHUNK + 16,), jnp.int32),
            pltpu.VMEM((CHUNK,), jnp.int32),
            pltpu.VMEM((CHUNK,), jnp.int32),
            pltpu.VMEM((CHUNK,), jnp.float32),
            pltpu.VMEM((CHUNK,), jnp.float32),
            pltpu.VMEM((CHUNK, HWORDS), jnp.int32),
            pltpu.VMEM((CHUNK, HWORDS), jnp.int32),
            pltpu.SemaphoreType.DMA,
            pltpu.SemaphoreType.DMA,
            pltpu.SemaphoreType.DMA,
            pltpu.SemaphoreType.DMA,
        ],
    )
    def body(h_hbm, a_hbm, b_hbm, src_hbm, dst_hbm, bounds_hbm, bias_hbm,
             out_hbm, stage, a_v, b_v, bias_v, bounds_v,
             src_a, src_b, dst_a, dst_b, sco_a, sco_b, rows_a, rows_b,
             sem_ia, sem_ib, sem_ra, sem_rb):
        wid = lax.axis_index("s") * NC + lax.axis_index("c")
        n_lo = wid * NODES_PER
        a_base = pl.multiple_of(jnp.maximum(n_lo - 32, 0), 8)

        pltpu.sync_copy(a_hbm.at[pl.ds(a_base, AWIN)], a_v)
        pltpu.sync_copy(b_hbm, b_v)
        pltpu.sync_copy(bias_hbm, bias_v)
        pltpu.sync_copy(bounds_hbm, bounds_v)

        iota = lax.broadcasted_iota(jnp.int32, (16,), 0)
        bitval = jnp.left_shift(jnp.ones((16,), jnp.int32), iota)
        n_lo_v = jnp.full((16,), n_lo, jnp.int32)
        n_hi_v = n_lo_v + NODES_PER
        a_base_v = jnp.full((16,), a_base, jnp.int32)
        cols = []
        for j8 in range(NBLK):
            cols.append(iota + 16 * j8)
            cols.append(iota + 128 + 16 * j8)

        def prefill(i, _):
            for j in range(JBLK):
                stage[i, pl.ds(j * 16, 16)] = bias_v[pl.ds(j * 16, 16)]
            return 0
        lax.fori_loop(0, NODES_PER, prefill, 0)

        widv = jnp.full((16,), wid, jnp.int32)
        e_lo = jnp.max(plsc.load_gather(bounds_v, [widv]))
        e_hi = jnp.max(plsc.load_gather(bounds_v, [widv + 1]))
        e0 = (e_lo // 8) * 8
        nchunks = (e_hi - e0 + (CHUNK - 1)) // CHUNK
        pairs = jnp.maximum((nchunks + 1) // 2, 1)

        def ids_start(c, src_ref, dst_ref, sem):
            e = pl.multiple_of(e0 + c * CHUNK, 8)
            pltpu.async_copy(src_hbm.at[pl.ds(e, CHUNK)],
                             src_ref.at[pl.ds(16, CHUNK)], sem)
            pltpu.async_copy(dst_hbm.at[pl.ds(e, CHUNK)], dst_ref, sem)

        def ids_wait(src_ref, dst_ref, sem):
            pltpu.make_async_copy(src_hbm.at[pl.ds(0, CHUNK)],
                                  src_ref.at[pl.ds(16, CHUNK)], sem).wait()
            pltpu.make_async_copy(dst_hbm.at[pl.ds(0, CHUNK)],
                                  dst_ref, sem).wait()

        def rows_start(dst_ref, rows_ref, sem):
            pltpu.async_copy(h_hbm.at[dst_ref], rows_ref, sem)

        def rows_wait(dst_ref, rows_ref, sem):
            pltpu.make_async_copy(h_hbm.at[dst_ref], rows_ref, sem).wait()

        def flush_stores(acc, accden, cur_src):
            nloc = cur_src - n_lo_v
            vmask = (cur_src >= n_lo_v) & (cur_src < n_hi_v)
            w = 1.0 / accden
            for j in range(JBLK):
                plsc.store_scatter(
                    stage, [nloc, cols[j]],
                    acc[j] * w + plsc.load_gather(bias_v, [cols[j]]),
                    mask=vmask)

        def process(src_ref, dst_ref, sco_ref, rows_ref, carry):
            for g in range(NGRP):
                s16 = src_ref[pl.ds(16 + g * 16, 16)]
                d16 = dst_ref[pl.ds(g * 16, 16)]
                prevv = plsc.load_gather(src_ref, [iota + (15 + g * 16)])
                aidx = jnp.clip(s16 - a_base_v, 0, AWIN - 1)
                x = plsc.load_gather(a_v, [aidx]) + plsc.load_gather(b_v, [d16])
                x = jnp.where(x >= 0.0, x, 0.2 * x)
                sco_ref[pl.ds(g * 16, 16)] = jnp.exp(x)
                bmask = jnp.sum(jnp.where(s16 != prevv, bitval, 0))

                def edge_body(i, car):
                    acc, accden, cur_src = car
                    idx = g * 16 + i
                    s_vec = plsc.load_gather(
                        sco_ref, [jnp.full((16,), idx, jnp.int32)])
                    contrib = []
                    for j8 in range(NBLK):
                        v = rows_ref[idx, pl.ds(j8 * 16, 16)]
                        lo = plsc.bitcast(jnp.left_shift(v, 16), jnp.float32)
                        hi = plsc.bitcast(
                            jnp.bitwise_and(v, jnp.int32(-65536)), jnp.float32)
                        contrib.append(s_vec * lo)
                        contrib.append(s_vec * hi)
                    bit = jnp.bitwise_and(
                        jnp.right_shift(bmask, i), jnp.int32(1))
                    flush = bit == 1

                    @pl.when(flush)
                    def _():
                        flush_stores(acc, accden, cur_src)

                    new_src = plsc.load_gather(
                        src_ref, [jnp.full((16,), 16 + idx, jnp.int32)])
                    acc2 = tuple(
                        jnp.where(flush, contrib[j], acc[j] + contrib[j])
                        for j in range(JBLK))
                    den2 = jnp.where(flush, s_vec, accden + s_vec)
                    cur2 = jnp.where(flush, new_src, cur_src)
                    return acc2, den2, cur2

                carry = lax.fori_loop(0, 16, edge_body, carry)
            return carry

        acc0 = tuple(jnp.zeros((16,), jnp.float32) for _ in range(JBLK))
        den0 = jnp.ones((16,), jnp.float32)
        src0 = jnp.full((16,), -1, jnp.int32)

        src_a[pl.ds(0, 16)] = jnp.full((16,), -1, jnp.int32)
        src_b[pl.ds(0, 16)] = jnp.full((16,), -1, jnp.int32)
        ids_start(0, src_a, dst_a, sem_ia)
        ids_wait(src_a, dst_a, sem_ia)
        rows_start(dst_a, rows_a, sem_ra)
        ids_start(1, src_b, dst_b, sem_ib)

        def pair_body(m, carry):
            ids_wait(src_b, dst_b, sem_ib)
            src_b[pl.ds(0, 16)] = src_a[pl.ds(CHUNK, 16)]
            rows_start(dst_b, rows_b, sem_rb)
            rows_wait(dst_a, rows_a, sem_ra)
            carry = process(src_a, dst_a, sco_a, rows_a, carry)

            @pl.when(m + 1 < pairs)
            def _():
                ids_start(2 * m + 2, src_a, dst_a, sem_ia)

            rows_wait(dst_b, rows_b, sem_rb)
            carry = process(src_b, dst_b, sco_b, rows_b, carry)

            @pl.when(m + 1 < pairs)
            def _():
                ids_wait(src_a, dst_a, sem_ia)
                src_a[pl.ds(0, 16)] = src_b[pl.ds(CHUNK, 16)]
                rows_start(dst_a, rows_a, sem_ra)
                ids_start(2 * m + 3, src_b, dst_b, sem_ib)

            return carry

        acc, accden, cur_src = lax.fori_loop(
            0, pairs, pair_body, (acc0, den0, src0))

        flush_stores(acc, accden, cur_src)

        last_rows = N_NODES - (NW - 1) * NODES_PER

        @pl.when(wid < NW - 1)
        def _():
            pltpu.sync_copy(stage, out_hbm.at[pl.ds(n_lo, NODES_PER)])

        @pl.when(wid == NW - 1)
        def _():
            pltpu.sync_copy(
                stage.at[pl.ds(0, last_rows)],
                out_hbm.at[pl.ds((NW - 1) * NODES_PER, last_rows)])

    return body(h32, a_p, b_p, src_p, dst_p, bounds_p, bias)


def kernel(node_states, edges, kernel, bias, kernel_attention, training):
    del training
    x = node_states.astype(jnp.float32)
    w = kernel.astype(jnp.float32)
    ka = kernel_attention.astype(jnp.float32)
    ka_pad = jnp.zeros((128, D_FEAT), jnp.float32)
    ka_pad = ka_pad.at[0].set(ka[:UNITS]).at[1].set(ka[UNITS:])

    h32, ab = _tc_transform(x, w, ka_pad)

    src = edges[:, 0].astype(jnp.int32)
    dst = edges[:, 1].astype(jnp.int32)
    src_p = jnp.concatenate(
        [src, jnp.full((EPAD - N_EDGES,), SRC_PAD, jnp.int32)])
    dst_p = jnp.concatenate(
        [dst, jnp.zeros((EPAD - N_EDGES,), jnp.int32)])
    bound_nodes = jnp.arange(33, dtype=jnp.int32) * NODES_PER
    bounds = jnp.searchsorted(src, bound_nodes).astype(jnp.int32)
    bounds_p = jnp.concatenate([bounds, jnp.zeros((7,), jnp.int32)])
    a_p = jnp.concatenate([ab[:, 0], jnp.zeros((APAD - N_NODES,), jnp.float32)])
    b_p = jnp.concatenate([ab[:, 1], jnp.zeros((NPAD - N_NODES,), jnp.float32)])

    return _sc_aggregate(h32, a_p, b_p, src_p, dst_p, bounds_p,
                         bias.astype(jnp.float32))
